# baseline (speedup 1.0000x reference)
import functools

import jax
import jax.numpy as jnp
from jax import lax
from jax.experimental import pallas as pl
from jax.experimental.pallas import tpu as pltpu
from jax.experimental.pallas import tpu_sc as plsc

BATCH = 4096
D = 64
N_IN = 1000000
NNZ_PAD = 104
NC = 2
NS = 16
NW = NC * NS
BPW = BATCH // NW
NBUF = 4
NCH = D // 16

TCOLS = 2048


def _tr_body(wt_ref, out_ref):
    x = wt_ref[...]
    eye = jnp.eye(D, dtype=jnp.float32)
    t = lax.dot_general(x, eye, (((0,), (0,)), ((), ())),
                        preferred_element_type=jnp.float32)
    t3 = jnp.reshape(t, (TCOLS // 2, 2, D))
    out_ref[...] = jnp.concatenate([t3[:, 0, :], t3[:, 1, :]], axis=1)


def _transpose(wt):
    grid = (N_IN + TCOLS - 1) // TCOLS
    return pl.pallas_call(
        _tr_body,
        grid=(grid,),
        in_specs=[pl.BlockSpec((D, TCOLS), lambda i: (0, i))],
        out_specs=pl.BlockSpec((TCOLS // 2, 2 * D), lambda i: (i, 0)),
        out_shape=jax.ShapeDtypeStruct((N_IN // 2, 2 * D), jnp.float32),
    )(wt)


def _build_gather():
    mesh = plsc.VectorSubcoreMesh(core_axis_name="c", subcore_axis_name="s")

    @functools.partial(
        pl.kernel,
        out_type=jax.ShapeDtypeStruct((BATCH, D), jnp.float32),
        mesh=mesh,
        compiler_params=pltpu.CompilerParams(use_tc_tiling_on_sc=False),
        scratch_types=[
            pltpu.VMEM((BPW, NNZ_PAD), jnp.int32),
            pltpu.VMEM((BPW, NNZ_PAD), jnp.float32),
            pltpu.VMEM((NBUF, NNZ_PAD, D), jnp.float32),
            pltpu.VMEM((BPW, D), jnp.float32),
            pltpu.VMEM((D,), jnp.float32),
            [pltpu.SemaphoreType.DMA] * NBUF,
        ],
    )
    def sc_fn(val_hbm, idx_hbm, w_hbm, bias_hbm, out_hbm,
              idx_v, val_v, rows_v, out_v, bias_v, sems):
        wid = lax.axis_index("s") * NC + lax.axis_index("c")
        base = wid * BPW
        pltpu.sync_copy(idx_hbm.at[pl.ds(base, BPW)], idx_v)
        pltpu.sync_copy(val_hbm.at[pl.ds(base, BPW)], val_v)
        pltpu.sync_copy(bias_hbm, bias_v)

        bias_regs = tuple(bias_v[pl.ds(c * 16, 16)] for c in range(NCH))

        def start(b, k):
            pltpu.async_copy(w_hbm.at[idx_v.at[b]], rows_v.at[k], sems[k])

        def wait(k):
            pltpu.make_async_copy(w_hbm.at[idx_v.at[0]], rows_v.at[k], sems[k]).wait()

        def fma(v, accs, k, j):
            return tuple(
                accs[c] + v * rows_v[k, j, pl.ds(c * 16, 16)]
                for c in range(NCH)
            )

        def compute(b, k):
            def jblock(i, accs):
                jj = i * 16
                vals16 = val_v[b, pl.ds(jj, 16)]
                for t in range(16):
                    accs = fma(vals16[t], accs, k, jj + t)
                return accs
            accs = lax.fori_loop(0, 6, jblock, bias_regs)
            vals16 = val_v[b, pl.ds(88, 16)]
            for t in range(8, 16):
                accs = fma(vals16[t], accs, k, 88 + t)
            for c in range(NCH):
                out_v[b, pl.ds(c * 16, 16)] = accs[c]

        for k in range(NBUF):
            start(k, k)

        @pl.loop(0, BPW, step=NBUF)
        def _body(bb):
            for k in range(NBUF):
                b = bb + k
                wait(k)
                compute(b, k)

                @pl.when(b + NBUF < BPW)
                def _():
                    start(b + NBUF, k)

        pltpu.sync_copy(out_v, out_hbm.at[pl.ds(base, BPW)])

    return sc_fn


_GATHER = _build_gather()

@jax.jit
def kernel(values, indices, weight, bias):
    nnz = values.shape[1]
    pad = NNZ_PAD - nnz
    val_p = jnp.pad(values, ((0, 0), (0, pad)))
    idx_p = jnp.pad(indices, ((0, 0), (0, pad)))
    wt = jnp.transpose(weight)
    w_lin2 = _transpose(wt)
    w_lin = jnp.reshape(w_lin2, (N_IN, D))
    return _GATHER(val_p, idx_p, w_lin, bias)


# speedup vs baseline: 1.2125x; 1.2125x over previous
import functools

import jax
import jax.numpy as jnp
from jax import lax
from jax.experimental import pallas as pl
from jax.experimental.pallas import tpu as pltpu
from jax.experimental.pallas import tpu_sc as plsc

BATCH = 4096
D = 64
N_IN = 1000000
NNZ_PAD = 104
NC = 2
NS = 16
NW = NC * NS
BPW = BATCH // NW
NBUF = 8
NCH = D // 16

TCOLS = 8192


def _tr_body(wt_ref, out_ref):
    x = wt_ref[...]
    t = jnp.transpose(x)
    t3 = jnp.reshape(t, (TCOLS // 2, 2, D))
    out_ref[...] = jnp.concatenate([t3[:, 0, :], t3[:, 1, :]], axis=1)


def _transpose(wt):
    grid = (N_IN + TCOLS - 1) // TCOLS
    return pl.pallas_call(
        _tr_body,
        grid=(grid,),
        in_specs=[pl.BlockSpec((D, TCOLS), lambda i: (0, i))],
        out_specs=pl.BlockSpec((TCOLS // 2, 2 * D), lambda i: (i, 0)),
        out_shape=jax.ShapeDtypeStruct((N_IN // 2, 2 * D), jnp.float32),
    )(wt)


def _build_gather():
    mesh = plsc.VectorSubcoreMesh(core_axis_name="c", subcore_axis_name="s")

    @functools.partial(
        pl.kernel,
        out_type=jax.ShapeDtypeStruct((BATCH, D), jnp.float32),
        mesh=mesh,
        compiler_params=pltpu.CompilerParams(use_tc_tiling_on_sc=False),
        scratch_types=[
            pltpu.VMEM((BPW, NNZ_PAD), jnp.int32),
            pltpu.VMEM((BPW, NNZ_PAD), jnp.float32),
            pltpu.VMEM((NBUF, NNZ_PAD, D), jnp.float32),
            pltpu.VMEM((BPW, D), jnp.float32),
            pltpu.VMEM((D,), jnp.float32),
            [pltpu.SemaphoreType.DMA] * NBUF,
        ],
    )
    def sc_fn(val_hbm, idx_hbm, w_hbm, bias_hbm, out_hbm,
              idx_v, val_v, rows_v, out_v, bias_v, sems):
        wid = lax.axis_index("s") * NC + lax.axis_index("c")
        base = wid * BPW
        pltpu.sync_copy(idx_hbm.at[pl.ds(base, BPW)], idx_v)
        pltpu.sync_copy(val_hbm.at[pl.ds(base, BPW)], val_v)
        pltpu.sync_copy(bias_hbm, bias_v)

        bias_regs = tuple(bias_v[pl.ds(c * 16, 16)] for c in range(NCH))

        def start(b, k):
            pltpu.async_copy(w_hbm.at[idx_v.at[b]], rows_v.at[k], sems[k])

        def wait(k):
            pltpu.make_async_copy(w_hbm.at[idx_v.at[0]], rows_v.at[k], sems[k]).wait()

        def fma(v, accs, k, j):
            return tuple(
                accs[c] + v * rows_v[k, j, pl.ds(c * 16, 16)]
                for c in range(NCH)
            )

        def compute(b, k):
            def jblock(i, accs):
                jj = i * 16
                vals16 = val_v[b, pl.ds(jj, 16)]
                for t in range(16):
                    accs = fma(vals16[t], accs, k, jj + t)
                return accs
            accs = lax.fori_loop(0, 6, jblock, bias_regs)
            vals16 = val_v[b, pl.ds(88, 16)]
            for t in range(8, 16):
                accs = fma(vals16[t], accs, k, 88 + t)
            for c in range(NCH):
                out_v[b, pl.ds(c * 16, 16)] = accs[c]

        for k in range(NBUF):
            start(k, k)

        @pl.loop(0, BPW, step=NBUF)
        def _body(bb):
            for k in range(NBUF):
                b = bb + k
                wait(k)
                compute(b, k)

                @pl.when(b + NBUF < BPW)
                def _():
                    start(b + NBUF, k)

        pltpu.sync_copy(out_v, out_hbm.at[pl.ds(base, BPW)])

    return sc_fn


_GATHER = _build_gather()

@jax.jit
def kernel(values, indices, weight, bias):
    nnz = values.shape[1]
    pad = NNZ_PAD - nnz
    val_p = jnp.pad(values, ((0, 0), (0, pad)))
    idx_p = jnp.pad(indices, ((0, 0), (0, pad)))
    wt = jnp.transpose(weight)
    w_lin2 = _transpose(wt)
    w_lin = jnp.reshape(w_lin2, (N_IN, D))
    return _GATHER(val_p, idx_p, w_lin, bias)


# speedup vs baseline: 1.2168x; 1.0036x over previous
import functools

import jax
import jax.numpy as jnp
from jax import lax
from jax.experimental import pallas as pl
from jax.experimental.pallas import tpu as pltpu
from jax.experimental.pallas import tpu_sc as plsc

BATCH = 4096
D = 64
N_IN = 1000000
NNZ_PAD = 104
NC = 2
NS = 16
NW = NC * NS
BPW = BATCH // NW
NBUF = 8
NCH = D // 16

TCOLS = 16384


def _tr_body(wt_ref, out_ref):
    x = wt_ref[...]
    t = jnp.transpose(x)
    t3 = jnp.reshape(t, (TCOLS // 2, 2, D))
    out_ref[...] = jnp.concatenate([t3[:, 0, :], t3[:, 1, :]], axis=1)


def _transpose(wt):
    grid = (N_IN + TCOLS - 1) // TCOLS
    return pl.pallas_call(
        _tr_body,
        grid=(grid,),
        in_specs=[pl.BlockSpec((D, TCOLS), lambda i: (0, i))],
        out_specs=pl.BlockSpec((TCOLS // 2, 2 * D), lambda i: (i, 0)),
        out_shape=jax.ShapeDtypeStruct((N_IN // 2, 2 * D), jnp.float32),
    )(wt)


def _build_gather():
    mesh = plsc.VectorSubcoreMesh(core_axis_name="c", subcore_axis_name="s")

    @functools.partial(
        pl.kernel,
        out_type=jax.ShapeDtypeStruct((BATCH, D), jnp.float32),
        mesh=mesh,
        compiler_params=pltpu.CompilerParams(use_tc_tiling_on_sc=False),
        scratch_types=[
            pltpu.VMEM((BPW, NNZ_PAD), jnp.int32),
            pltpu.VMEM((BPW, NNZ_PAD), jnp.float32),
            pltpu.VMEM((NBUF, NNZ_PAD, D), jnp.float32),
            pltpu.VMEM((BPW, D), jnp.float32),
            pltpu.VMEM((D,), jnp.float32),
            [pltpu.SemaphoreType.DMA] * NBUF,
        ],
    )
    def sc_fn(val_hbm, idx_hbm, w_hbm, bias_hbm, out_hbm,
              idx_v, val_v, rows_v, out_v, bias_v, sems):
        wid = lax.axis_index("s") * NC + lax.axis_index("c")
        base = wid * BPW
        pltpu.sync_copy(idx_hbm.at[pl.ds(base, BPW)], idx_v)
        pltpu.sync_copy(val_hbm.at[pl.ds(base, BPW)], val_v)
        pltpu.sync_copy(bias_hbm, bias_v)

        bias_regs = tuple(bias_v[pl.ds(c * 16, 16)] for c in range(NCH))

        def start(b, k):
            pltpu.async_copy(w_hbm.at[idx_v.at[b]], rows_v.at[k], sems[k])

        def wait(k):
            pltpu.make_async_copy(w_hbm.at[idx_v.at[0]], rows_v.at[k], sems[k]).wait()

        def fma(v, accs, k, j):
            return tuple(
                accs[c] + v * rows_v[k, j, pl.ds(c * 16, 16)]
                for c in range(NCH)
            )

        def compute(b, k):
            def jblock(i, accs):
                jj = i * 16
                vals16 = val_v[b, pl.ds(jj, 16)]
                for t in range(16):
                    accs = fma(vals16[t], accs, k, jj + t)
                return accs
            accs = lax.fori_loop(0, 6, jblock, bias_regs)
            vals16 = val_v[b, pl.ds(88, 16)]
            for t in range(8, 16):
                accs = fma(vals16[t], accs, k, 88 + t)
            for c in range(NCH):
                out_v[b, pl.ds(c * 16, 16)] = accs[c]

        for k in range(NBUF):
            start(k, k)

        @pl.loop(0, BPW, step=NBUF)
        def _body(bb):
            for k in range(NBUF):
                b = bb + k
                wait(k)
                compute(b, k)

                @pl.when(b + NBUF < BPW)
                def _():
                    start(b + NBUF, k)

        pltpu.sync_copy(out_v, out_hbm.at[pl.ds(base, BPW)])

    return sc_fn


_GATHER = _build_gather()

@jax.jit
def kernel(values, indices, weight, bias):
    nnz = values.shape[1]
    pad = NNZ_PAD - nnz
    val_p = jnp.pad(values, ((0, 0), (0, pad)))
    idx_p = jnp.pad(indices, ((0, 0), (0, pad)))
    wt = jnp.transpose(weight)
    w_lin2 = _transpose(wt)
    w_lin = jnp.reshape(w_lin2, (N_IN, D))
    return _GATHER(val_p, idx_p, w_lin, bias)


# speedup vs baseline: 1.2197x; 1.0024x over previous
import functools

import jax
import jax.numpy as jnp
from jax import lax
from jax.experimental import pallas as pl
from jax.experimental.pallas import tpu as pltpu
from jax.experimental.pallas import tpu_sc as plsc

BATCH = 4096
D = 64
N_IN = 1000000
NNZ_PAD = 104
NC = 2
NS = 16
NW = NC * NS
BPW = BATCH // NW
NBUF = 8
NCH = D // 16

TCOLS = 16384


def _tr_body(wt_ref, out_ref):
    x = wt_ref[...]
    t = jnp.transpose(x)
    t3 = jnp.reshape(t, (TCOLS // 2, 2, D))
    out_ref[...] = jnp.concatenate([t3[:, 0, :], t3[:, 1, :]], axis=1)


def _transpose(wt):
    grid = (N_IN + TCOLS - 1) // TCOLS
    return pl.pallas_call(
        _tr_body,
        grid=(grid,),
        in_specs=[pl.BlockSpec((D, TCOLS), lambda i: (0, i))],
        out_specs=pl.BlockSpec((TCOLS // 2, 2 * D), lambda i: (i, 0)),
        out_shape=jax.ShapeDtypeStruct((N_IN // 2, 2 * D), jnp.float32),
    )(wt)


def _build_gather():
    mesh = plsc.VectorSubcoreMesh(core_axis_name="c", subcore_axis_name="s")

    @functools.partial(
        pl.kernel,
        out_type=jax.ShapeDtypeStruct((BATCH, D), jnp.float32),
        mesh=mesh,
        compiler_params=pltpu.CompilerParams(use_tc_tiling_on_sc=False),
        scratch_types=[
            pltpu.VMEM((BPW, NNZ_PAD), jnp.int32),
            pltpu.VMEM((BPW, NNZ_PAD), jnp.float32),
            pltpu.VMEM((NBUF, NNZ_PAD, D), jnp.float32),
            pltpu.VMEM((BPW, D), jnp.float32),
            pltpu.VMEM((D,), jnp.float32),
            [pltpu.SemaphoreType.DMA] * NBUF,
        ],
    )
    def sc_fn(val_hbm, idx_hbm, w_hbm, bias_hbm, out_hbm,
              idx_v, val_v, rows_v, out_v, bias_v, sems):
        wid = lax.axis_index("s") * NC + lax.axis_index("c")
        base = wid * BPW
        pltpu.sync_copy(idx_hbm.at[pl.ds(base, BPW)], idx_v)
        pltpu.sync_copy(val_hbm.at[pl.ds(base, BPW)], val_v)
        pltpu.sync_copy(bias_hbm, bias_v)

        bias_regs = tuple(bias_v[pl.ds(c * 16, 16)] for c in range(NCH))

        H1 = 56

        def start(b, k):
            pltpu.async_copy(
                w_hbm.at[idx_v.at[b, pl.ds(0, H1)]],
                rows_v.at[k, pl.ds(0, H1)], sems[k])
            pltpu.async_copy(
                w_hbm.at[idx_v.at[b, pl.ds(H1, NNZ_PAD - H1)]],
                rows_v.at[k, pl.ds(H1, NNZ_PAD - H1)], sems[k])

        def wait(k):
            pltpu.make_async_copy(
                w_hbm.at[idx_v.at[0, pl.ds(0, H1)]],
                rows_v.at[k, pl.ds(0, H1)], sems[k]).wait()
            pltpu.make_async_copy(
                w_hbm.at[idx_v.at[0, pl.ds(H1, NNZ_PAD - H1)]],
                rows_v.at[k, pl.ds(H1, NNZ_PAD - H1)], sems[k]).wait()

        def fma(v, accs, k, j):
            return tuple(
                accs[c] + v * rows_v[k, j, pl.ds(c * 16, 16)]
                for c in range(NCH)
            )

        def compute(b, k):
            def jblock(i, accs):
                jj = i * 16
                vals16 = val_v[b, pl.ds(jj, 16)]
                for t in range(16):
                    accs = fma(vals16[t], accs, k, jj + t)
                return accs
            accs = lax.fori_loop(0, 6, jblock, bias_regs)
            vals16 = val_v[b, pl.ds(88, 16)]
            for t in range(8, 16):
                accs = fma(vals16[t], accs, k, 88 + t)
            for c in range(NCH):
                out_v[b, pl.ds(c * 16, 16)] = accs[c]

        for k in range(NBUF):
            start(k, k)

        @pl.loop(0, BPW, step=NBUF)
        def _body(bb):
            for k in range(NBUF):
                b = bb + k
                wait(k)
                compute(b, k)

                @pl.when(b + NBUF < BPW)
                def _():
                    start(b + NBUF, k)

        pltpu.sync_copy(out_v, out_hbm.at[pl.ds(base, BPW)])

    return sc_fn


_GATHER = _build_gather()

@jax.jit
def kernel(values, indices, weight, bias):
    nnz = values.shape[1]
    pad = NNZ_PAD - nnz
    val_p = jnp.pad(values, ((0, 0), (0, pad)))
    idx_p = jnp.pad(indices, ((0, 0), (0, pad)))
    wt = jnp.transpose(weight)
    w_lin2 = _transpose(wt)
    w_lin = jnp.reshape(w_lin2, (N_IN, D))
    return _GATHER(val_p, idx_p, w_lin, bias)
